# baseline (speedup 1.0000x reference)
import jax
import jax.numpy as jnp
from jax import lax
from jax.experimental import pallas as pl
from jax.experimental.pallas import tpu as pltpu

_N = 4194304
_LANES = 384
_R = (_N * 3) // _LANES
_BR = 1024


def _tc_body(w_ref, b_ref, p_ref, x_ref, o_ref):
    blk = x_ref[...]
    s1 = pltpu.roll(blk, _LANES - 1, 1)
    s2 = pltpu.roll(blk, _LANES - 2, 1)
    mn = jnp.minimum(jnp.minimum(blk, s1), s2)
    mx = jnp.maximum(jnp.maximum(blk, s1), s2)
    sm = blk + s1 + s2
    w0 = w_ref[0, 0]
    w1 = w_ref[0, 1]
    w2 = w_ref[0, 2]
    r = sm * w1 + mn * (w0 - w1) + mx * (w2 - w1)
    out = lax.dot_general(
        r, p_ref[...],
        dimension_numbers=(((1,), (0,)), ((), ())),
        preferred_element_type=jnp.float32,
        precision=lax.Precision.HIGHEST,
    )
    o_ref[...] = out + b_ref[0]


def kernel(x, W, b):
    x2d = x.reshape(_R, _LANES)
    p = (jnp.arange(_LANES)[:, None] == 3 * jnp.arange(128)[None, :]).astype(jnp.float32)
    out = pl.pallas_call(
        _tc_body,
        grid=(_R // _BR,),
        in_specs=[
            pl.BlockSpec(memory_space=pltpu.SMEM),
            pl.BlockSpec(memory_space=pltpu.SMEM),
            pl.BlockSpec((_LANES, 128), lambda i: (0, 0)),
            pl.BlockSpec((_BR, _LANES), lambda i: (i, 0)),
        ],
        out_specs=pl.BlockSpec((_BR, 128), lambda i: (i, 0)),
        out_shape=jax.ShapeDtypeStruct((_R, 128), jnp.float32),
    )(W, b, p, x2d)
    return out.reshape(_N, 1)
